# baseline (speedup 1.0000x reference)
import functools

import jax
import jax.numpy as jnp
from jax import lax
from jax.experimental import pallas as pl
from jax.experimental.pallas import tpu as pltpu
from jax.experimental.pallas import tpu_sc as plsc

_HIDDEN = 768
_EXPERTS = 8
_SOFTCAP = 30.0
_RBLK = 2048


def _gate_tc_body(x_ref, w_ref, out_ref):
    logits = lax.dot_general(
        w_ref[...], x_ref[...],
        (((1,), (1,)), ((), ())),
        preferred_element_type=jnp.float32,
    )
    out_ref[...] = jnp.tanh(logits * (1.0 / _SOFTCAP)) * _SOFTCAP


def _gate_logits_t(x2d, gate_w):
    n = x2d.shape[0]
    return pl.pallas_call(
        _gate_tc_body,
        grid=(n // _RBLK,),
        in_specs=[
            pl.BlockSpec((_RBLK, _HIDDEN), lambda i: (i, 0)),
            pl.BlockSpec((_EXPERTS, _HIDDEN), lambda i: (0, 0)),
        ],
        out_specs=pl.BlockSpec((_EXPERTS, _RBLK), lambda i: (0, i)),
        out_shape=jax.ShapeDtypeStruct((_EXPERTS, n), jnp.float32),
    )(x2d, gate_w)


def _routing_sc(logits_t):
    n = logits_t.shape[1]
    info = plsc.get_sparse_core_info()
    num_cores, num_subcores, lanes = (
        info.num_cores, info.num_subcores, info.num_lanes)
    workers = num_cores * num_subcores
    per_w = n // workers
    mesh = plsc.VectorSubcoreMesh(core_axis_name="c", subcore_axis_name="s")

    @functools.partial(
        pl.kernel,
        mesh=mesh,
        compiler_params=pltpu.CompilerParams(needs_layout_passes=False),
        out_type=[
            jax.ShapeDtypeStruct((2 * n,), jnp.float32),
            jax.ShapeDtypeStruct((2 * n,), jnp.int32),
        ],
        scratch_types=[
            pltpu.VMEM((_EXPERTS, per_w), jnp.float32),
            pltpu.VMEM((2 * per_w,), jnp.float32),
            pltpu.VMEM((2 * per_w,), jnp.int32),
        ],
    )
    def k(l_hbm, w_hbm, e_hbm, lv, wv, ev):
        wid = lax.axis_index("s") * num_cores + lax.axis_index("c")
        base = wid * per_w
        pltpu.sync_copy(l_hbm.at[:, pl.ds(base, per_w)], lv)
        lane_ids = lax.iota(jnp.int32, lanes)

        def body(g, carry):
            off = g * lanes
            pos0 = 2 * (off + lane_ids)
            pos1 = pos0 + 1
            m1 = lv[0, pl.ds(off, lanes)]
            i1 = jnp.zeros((lanes,), jnp.int32)
            m2 = jnp.full((lanes,), -1e30, jnp.float32)
            i2 = jnp.zeros((lanes,), jnp.int32)
            for e in range(1, _EXPERTS):
                le = lv[e, pl.ds(off, lanes)]
                ei = jnp.full((lanes,), e, jnp.int32)
                gt1 = le > m1
                gt2 = le > m2
                m2 = jnp.where(gt1, m1, jnp.where(gt2, le, m2))
                i2 = jnp.where(gt1, i1, jnp.where(gt2, ei, i2))
                m1 = jnp.where(gt1, le, m1)
                i1 = jnp.where(gt1, ei, i1)
            ed = jnp.exp(m2 - m1)
            s = ed + 1.0
            plsc.store_scatter(wv, [pos0], 1.0 / s)
            plsc.store_scatter(wv, [pos1], ed / s)
            plsc.store_scatter(ev, [pos0], i1)
            plsc.store_scatter(ev, [pos1], i2)
            return carry

        lax.fori_loop(0, per_w // lanes, body, 0)
        pltpu.sync_copy(wv, w_hbm.at[pl.ds(2 * base, 2 * per_w)])
        pltpu.sync_copy(ev, e_hbm.at[pl.ds(2 * base, 2 * per_w)])

    rw, se = k(logits_t)
    return rw.reshape(n, 2), se.reshape(n, 2)


def kernel(hidden_states, gate_w):
    b, s, h = hidden_states.shape
    x2d = hidden_states.reshape(-1, h)
    logits_t = _gate_logits_t(x2d, gate_w)
    return _routing_sc(logits_t)


# speedup vs baseline: 2.0492x; 2.0492x over previous
import functools

import jax
import jax.numpy as jnp
from jax import lax
from jax.experimental import pallas as pl
from jax.experimental.pallas import tpu as pltpu
from jax.experimental.pallas import tpu_sc as plsc

_HIDDEN = 768
_EXPERTS = 8
_SOFTCAP = 30.0
_RBLK = 2048
_SC_FRAC_BLKS = 6


def _gate_tc_body(x_ref, w_ref, out_ref):
    logits = lax.dot_general(
        w_ref[...], x_ref[...],
        (((1,), (1,)), ((), ())),
        preferred_element_type=jnp.float32,
    )
    out_ref[...] = jnp.tanh(logits * (1.0 / _SOFTCAP)) * _SOFTCAP


def _gate_logits_t(x2d, gate_w, nblk):
    return pl.pallas_call(
        _gate_tc_body,
        grid=(nblk,),
        in_specs=[
            pl.BlockSpec((_RBLK, _HIDDEN), lambda i: (i, 0)),
            pl.BlockSpec((_EXPERTS, _HIDDEN), lambda i: (0, 0)),
        ],
        out_specs=pl.BlockSpec((_EXPERTS, _RBLK), lambda i: (0, i)),
        out_shape=jax.ShapeDtypeStruct((_EXPERTS, nblk * _RBLK), jnp.float32),
    )(x2d, gate_w)


def _top2(l, sidx):
    m1 = jnp.max(l, axis=0, keepdims=True)
    i1 = jnp.min(jnp.where(l == m1, sidx, _EXPERTS), axis=0, keepdims=True)
    lm = jnp.where(sidx == i1, -1e9, l)
    m2 = jnp.max(lm, axis=0, keepdims=True)
    i2 = jnp.min(jnp.where(lm == m2, sidx, _EXPERTS), axis=0, keepdims=True)
    ed = jnp.exp(m2 - m1)
    w1 = 1.0 / (1.0 + ed)
    w2 = ed * w1
    return w1, w2, i1, i2


def _fused_tc_body(x_ref, w_ref, ow_ref, oi_ref):
    logits = lax.dot_general(
        w_ref[...], x_ref[...],
        (((1,), (1,)), ((), ())),
        preferred_element_type=jnp.float32,
    )
    l = jnp.tanh(logits * (1.0 / _SOFTCAP)) * _SOFTCAP
    sidx = lax.broadcasted_iota(jnp.int32, l.shape, 0)
    w1, w2, i1, i2 = _top2(l, sidx)
    ow_ref[...] = jnp.concatenate([w1, w2], axis=0)
    oi_ref[...] = jnp.concatenate([i1, i2], axis=0)


def _fused_tc(x2d, gate_w, blk0, nblk):
    return pl.pallas_call(
        _fused_tc_body,
        grid=(nblk,),
        in_specs=[
            pl.BlockSpec((_RBLK, _HIDDEN), lambda i: (i + blk0, 0)),
            pl.BlockSpec((_EXPERTS, _HIDDEN), lambda i: (0, 0)),
        ],
        out_specs=[
            pl.BlockSpec((2, _RBLK), lambda i: (0, i)),
            pl.BlockSpec((2, _RBLK), lambda i: (0, i)),
        ],
        out_shape=[
            jax.ShapeDtypeStruct((2, nblk * _RBLK), jnp.float32),
            jax.ShapeDtypeStruct((2, nblk * _RBLK), jnp.int32),
        ],
    )(x2d, gate_w)


def _routing_sc(logits_t):
    n = logits_t.shape[1]
    info = plsc.get_sparse_core_info()
    num_cores, num_subcores, lanes = (
        info.num_cores, info.num_subcores, info.num_lanes)
    workers = num_cores * num_subcores
    per_w = n // workers
    mesh = plsc.VectorSubcoreMesh(core_axis_name="c", subcore_axis_name="s")

    @functools.partial(
        pl.kernel,
        mesh=mesh,
        compiler_params=pltpu.CompilerParams(needs_layout_passes=False),
        out_type=[
            jax.ShapeDtypeStruct((2, n), jnp.float32),
            jax.ShapeDtypeStruct((2, n), jnp.int32),
        ],
        scratch_types=[
            pltpu.VMEM((_EXPERTS, per_w), jnp.float32),
            pltpu.VMEM((2, per_w), jnp.float32),
            pltpu.VMEM((2, per_w), jnp.int32),
        ],
    )
    def k(l_hbm, w_hbm, e_hbm, lv, wv, ev):
        wid = lax.axis_index("s") * num_cores + lax.axis_index("c")
        base = wid * per_w
        pltpu.sync_copy(l_hbm.at[:, pl.ds(base, per_w)], lv)

        def body(g, carry):
            off = g * lanes
            m1 = lv[0, pl.ds(off, lanes)]
            i1 = jnp.zeros((lanes,), jnp.int32)
            m2 = jnp.full((lanes,), -1e30, jnp.float32)
            i2 = jnp.zeros((lanes,), jnp.int32)
            for e in range(1, _EXPERTS):
                le = lv[e, pl.ds(off, lanes)]
                ei = jnp.full((lanes,), e, jnp.int32)
                gt1 = le > m1
                gt2 = le > m2
                m2 = jnp.where(gt1, m1, jnp.where(gt2, le, m2))
                i2 = jnp.where(gt1, i1, jnp.where(gt2, ei, i2))
                m1 = jnp.where(gt1, le, m1)
                i1 = jnp.where(gt1, ei, i1)
            ed = jnp.exp(m2 - m1)
            s = ed + 1.0
            wv[0, pl.ds(off, lanes)] = 1.0 / s
            wv[1, pl.ds(off, lanes)] = ed / s
            ev[0, pl.ds(off, lanes)] = i1
            ev[1, pl.ds(off, lanes)] = i2
            return carry

        lax.fori_loop(0, per_w // lanes, body, 0)
        pltpu.sync_copy(wv, w_hbm.at[:, pl.ds(base, per_w)])
        pltpu.sync_copy(ev, e_hbm.at[:, pl.ds(base, per_w)])

    rw, se = k(logits_t)
    return rw, se


def kernel(hidden_states, gate_w):
    b, s, h = hidden_states.shape
    x2d = hidden_states.reshape(-1, h)
    n = x2d.shape[0]
    nblk = n // _RBLK
    a_blks = _SC_FRAC_BLKS
    na = a_blks * _RBLK

    logits_a = _gate_logits_t(x2d, gate_w, a_blks)
    rw_a, se_a = _routing_sc(logits_a)
    rw_b, se_b = _fused_tc(x2d, gate_w, a_blks, nblk - a_blks)

    rw = jnp.concatenate([rw_a, rw_b], axis=1).T
    se = jnp.concatenate([se_a, se_b], axis=1).T
    return rw, se
